# baseline (speedup 1.0000x reference)
import jax
import jax.numpy as jnp
from jax import lax
from jax.experimental import pallas as pl
from jax.experimental.pallas import tpu as pltpu
from jax.experimental.pallas import tpu_sc as plsc

N = 50000
E = 800000
F = 111
H = 64
HH = H // 2

NC = 2
NS = 16

RPA = 3128
ACCA = NS * RPA
LASTA = N - 15 * RPA

HALF = N // NC
RPS = 1568
ACC = NS * RPS
LAST = HALF - 15 * RPS
DW = 8

C = 128
K = 3
GE = K * C
EPS = E // NS
NG = EPS // GE
TAIL = EPS - NG * GE

_MESH = plsc.VectorSubcoreMesh(core_axis_name="c", subcore_axis_name="s")
_SC_PARAMS = pltpu.CompilerParams(use_tc_tiling_on_sc=False)


def _agg_body(src_hbm, dst_hbm, h2_hbm, zeros_hbm, agg_hbm,
              acc, srcw, dstw, rows, isem0, isem1, *sems):
    gsem, ssem = sems[0:K], sems[K:2 * K]
    c = lax.axis_index("c")
    s = lax.axis_index("s")
    junk = N + s
    rowbase = c * N
    ebase = s * EPS

    pltpu.sync_copy(zeros_hbm.at[pl.ds(0, RPA)], acc.at[pl.ds(s * RPA, RPA)])
    plsc.subcore_barrier()


    def load_idx(g, p, sem):
        gbase = ebase + g * GE
        pltpu.async_copy(src_hbm.at[pl.ds(gbase, GE)], srcw.at[p], sem)
        for j in range(K):
            pltpu.async_copy(dst_hbm.at[pl.ds(gbase + j * C, C)],
                             dstw.at[lax.rem(g, 3), j], sem)

    def wait_idx(g, p, sem):
        gbase = ebase + g * GE
        pltpu.make_async_copy(src_hbm.at[pl.ds(gbase, GE)], srcw.at[p],
                              sem).wait()
        for j in range(K):
            pltpu.make_async_copy(dst_hbm.at[pl.ds(gbase + j * C, C)],
                                  dstw.at[lax.rem(g, 3), j], sem).wait()

    def rebase(p):
        for q in range(GE // 16):
            v = srcw[p, pl.ds(q * 16, 16)]
            srcw[p, pl.ds(q * 16, 16)] = v + rowbase

    def fire_gathers(p):
        for j in range(K):
            pltpu.async_copy(h2_hbm.at[srcw.at[p, pl.ds(j * C, C)]],
                             rows.at[p, j], gsem[j])

    load_idx(0, 0, isem0)
    wait_idx(0, 0, isem0)
    rebase(0)
    fire_gathers(0)

    def rnd(g, carry):
        p = lax.rem(g, 2)
        pn = lax.rem(g + 1, 2)
        w3 = lax.rem(g, 3)
        w3p = lax.rem(g + 2, 3)

        @pl.when(g >= 1)
        def _():
            for j in range(K):
                pltpu.make_async_copy(rows.at[pn, j], acc.at[dstw.at[w3p, j]],
                                      ssem[j]).wait()

        @pl.when(jnp.logical_and(g + 1 < NG, pn == 0))
        def _():
            load_idx(g + 1, 0, isem0)

        @pl.when(jnp.logical_and(g + 1 < NG, pn == 1))
        def _():
            load_idx(g + 1, 1, isem1)

        for j in range(K):
            pltpu.make_async_copy(h2_hbm.at[srcw.at[p, pl.ds(j * C, C)]],
                                  rows.at[p, j], gsem[j]).wait()
            pltpu.async_copy(rows.at[p, j], acc.at[dstw.at[w3, j]], ssem[j],
                             add=True)

        @pl.when(jnp.logical_and(g + 1 < NG, pn == 0))
        def _():
            wait_idx(g + 1, 0, isem0)

        @pl.when(jnp.logical_and(g + 1 < NG, pn == 1))
        def _():
            wait_idx(g + 1, 1, isem1)

        @pl.when(g + 1 < NG)
        def _():
            rebase(pn)
            fire_gathers(pn)

        return carry

    lax.fori_loop(0, NG, rnd, 0)

    for j in range(K):
        pltpu.make_async_copy(rows.at[(NG - 1) % 2, j],
                              acc.at[dstw.at[(NG - 1) % 3, j]],
                              ssem[j]).wait()

    toff = ebase + NG * GE
    pltpu.sync_copy(src_hbm.at[pl.ds(toff, TAIL)], srcw.at[0, pl.ds(0, TAIL)])
    pltpu.sync_copy(dst_hbm.at[pl.ds(toff, TAIL)],
                    dstw.at[0, 0, pl.ds(0, TAIL)])
    zero16 = jnp.zeros((16,), jnp.int32)
    for q in range(TAIL // 16, C // 16):
        srcw[0, pl.ds(q * 16, 16)] = zero16
        dstw[0, 0, pl.ds(q * 16, 16)] = zero16 + junk
    for q in range(C // 16):
        v = srcw[0, pl.ds(q * 16, 16)]
        srcw[0, pl.ds(q * 16, 16)] = v + rowbase
    pltpu.async_copy(h2_hbm.at[srcw.at[0, pl.ds(0, C)]], rows.at[0, 0],
                     gsem[0]).wait()
    pltpu.sync_copy(rows.at[0, 0], acc.at[dstw.at[0, 0]], add=True)

    plsc.subcore_barrier()

    @pl.when(s < NS - 1)
    def _():
        pltpu.sync_copy(acc.at[pl.ds(s * RPA, RPA)],
                        agg_hbm.at[pl.ds(c * N + s * RPA, RPA)])

    @pl.when(s == NS - 1)
    def _():
        pltpu.sync_copy(acc.at[pl.ds((NS - 1) * RPA, LASTA)],
                        agg_hbm.at[pl.ds(c * N + (NS - 1) * RPA, LASTA)])


_agg = pl.kernel(
    _agg_body,
    out_type=(jax.ShapeDtypeStruct((NC * N, HH), jnp.float32),),
    mesh=_MESH,
    scratch_types=(
        pltpu.VMEM_SHARED((ACCA, HH), jnp.float32),
        pltpu.VMEM((2, GE), jnp.int32),
        pltpu.VMEM((3, K, C), jnp.int32),
        pltpu.VMEM((2, K, C, HH), jnp.float32),
    ) + (pltpu.SemaphoreType.DMA,) * (2 + 2 * K),
    compiler_params=_SC_PARAMS,
)


def _deg_body(dst_hbm, zeros16_hbm, ones_hbm, deg_hbm,
              dacc, dstw, dloc, onesv, isem0, isem1, *ssem):
    c = lax.axis_index("c")
    s = lax.axis_index("s")
    base_node = c * HALF
    junk = HALF + s
    ebase = s * EPS

    pltpu.sync_copy(zeros16_hbm, dacc.at[pl.ds(s * RPS, RPS)])
    pltpu.sync_copy(ones_hbm, onesv)
    plsc.subcore_barrier()

    pltpu.async_copy(dst_hbm.at[pl.ds(ebase, GE)], dstw.at[0], isem0)

    def compute_dloc(p, j, nvec):
        for q in range(nvec):
            d = dstw[p, pl.ds(j * C + q * 16, 16)]
            loc = d - base_node
            ok = (loc >= 0) & (loc < HALF)
            dloc[j, pl.ds(q * 16, 16)] = jnp.where(ok, loc, junk)

    def rnd(g, carry):
        gbase = ebase + g * GE
        p = lax.rem(g, 2)

        @pl.when(jnp.logical_and(g + 1 < NG, p == 0))
        def _():
            pltpu.async_copy(dst_hbm.at[pl.ds(gbase + GE, GE)], dstw.at[1],
                             isem1)

        @pl.when(jnp.logical_and(g + 1 < NG, p == 1))
        def _():
            pltpu.async_copy(dst_hbm.at[pl.ds(gbase + GE, GE)], dstw.at[0],
                             isem0)

        @pl.when(g >= 1)
        def _():
            for j in range(K):
                pltpu.make_async_copy(onesv, dacc.at[dloc.at[j]],
                                      ssem[j]).wait()

        @pl.when(p == 0)
        def _():
            pltpu.make_async_copy(dst_hbm.at[pl.ds(gbase, GE)], dstw.at[0],
                                  isem0).wait()

        @pl.when(p == 1)
        def _():
            pltpu.make_async_copy(dst_hbm.at[pl.ds(gbase, GE)], dstw.at[1],
                                  isem1).wait()

        for j in range(K):
            compute_dloc(p, j, C // 16)
        for j in range(K):
            pltpu.async_copy(onesv, dacc.at[dloc.at[j]], ssem[j], add=True)
        return carry

    lax.fori_loop(0, NG, rnd, 0)
    for j in range(K):
        pltpu.make_async_copy(onesv, dacc.at[dloc.at[j]], ssem[j]).wait()

    toff = ebase + NG * GE
    pltpu.sync_copy(dst_hbm.at[pl.ds(toff, TAIL)], dstw.at[0, pl.ds(0, TAIL)])
    compute_dloc(0, 0, TAIL // 16)
    zero16 = jnp.zeros((16,), jnp.int32)
    for q in range(TAIL // 16, C // 16):
        dloc[0, pl.ds(q * 16, 16)] = zero16 + junk
    pltpu.sync_copy(onesv, dacc.at[dloc.at[0]], add=True)

    plsc.subcore_barrier()

    @pl.when(s < NS - 1)
    def _():
        pltpu.sync_copy(dacc.at[pl.ds(s * RPS, RPS)],
                        deg_hbm.at[pl.ds(c * HALF + s * RPS, RPS)])

    @pl.when(s == NS - 1)
    def _():
        pltpu.sync_copy(dacc.at[pl.ds((NS - 1) * RPS, LAST)],
                        deg_hbm.at[pl.ds(c * HALF + (NS - 1) * RPS, LAST)])


_deg = pl.kernel(
    _deg_body,
    out_type=(jax.ShapeDtypeStruct((N, DW), jnp.float32),),
    mesh=_MESH,
    scratch_types=(
        pltpu.VMEM_SHARED((ACC, DW), jnp.float32),
        pltpu.VMEM((2, GE), jnp.int32),
        pltpu.VMEM((K, C), jnp.int32),
        pltpu.VMEM((C, DW), jnp.float32),
    ) + (pltpu.SemaphoreType.DMA,) * (2 + K),
    compiler_params=_SC_PARAMS,
)

_BN = 2000


def _embed_body(x_ref, w_ref, b_ref, o_ref):
    hn = jnp.dot(x_ref[...], w_ref[...],
                 preferred_element_type=jnp.float32) + b_ref[...]
    o_ref[0] = hn[:, :HH]
    o_ref[1] = hn[:, HH:]


def _embed(x, w, b):
    return pl.pallas_call(
        _embed_body,
        grid=(N // _BN,),
        in_specs=[
            pl.BlockSpec((_BN, F), lambda i: (i, 0)),
            pl.BlockSpec((F, H), lambda i: (0, 0)),
            pl.BlockSpec((1, H), lambda i: (0, 0)),
        ],
        out_specs=pl.BlockSpec((2, _BN, HH), lambda i: (0, i, 0)),
        out_shape=jax.ShapeDtypeStruct((2, N, HH), jnp.float32),
    )(x, w, b)


def _update_body(h2_ref, agg2_ref, deg_ref, w_ref, b_ref, o_ref, cs_ref):
    i = pl.program_id(0)
    denom = jnp.maximum(deg_ref[:, 0:1], 1.0)
    h = jnp.concatenate([h2_ref[0], h2_ref[1]], axis=1)
    agg = jnp.concatenate([agg2_ref[0], agg2_ref[1]], axis=1)
    hn = jnp.maximum(
        jnp.dot(h + agg / denom, w_ref[...],
                preferred_element_type=jnp.float32) + b_ref[...], 0.0)
    o_ref[0] = hn[:, :HH]
    o_ref[1] = hn[:, HH:]

    @pl.when(i == 0)
    def _():
        cs_ref[...] = jnp.zeros_like(cs_ref)

    cs_ref[...] += jnp.sum(hn, axis=0, keepdims=True)


def _update(h2, agg2, deg, w, b):
    return pl.pallas_call(
        _update_body,
        grid=(N // _BN,),
        in_specs=[
            pl.BlockSpec((2, _BN, HH), lambda i: (0, i, 0)),
            pl.BlockSpec((2, _BN, HH), lambda i: (0, i, 0)),
            pl.BlockSpec((_BN, DW), lambda i: (i, 0)),
            pl.BlockSpec((H, H), lambda i: (0, 0)),
            pl.BlockSpec((1, H), lambda i: (0, 0)),
        ],
        out_specs=[
            pl.BlockSpec((2, _BN, HH), lambda i: (0, i, 0)),
            pl.BlockSpec((1, H), lambda i: (0, 0)),
        ],
        out_shape=[
            jax.ShapeDtypeStruct((2, N, HH), jnp.float32),
            jax.ShapeDtypeStruct((1, H), jnp.float32),
        ],
    )(h2, agg2, deg, w, b)


def _update_head_body(h2_ref, agg2_ref, deg_ref, w_ref, b_ref,
                      w1_ref, b1_ref, w2_ref, b2_ref, o_ref, cs_ref):
    i = pl.program_id(0)
    denom = jnp.maximum(deg_ref[:, 0:1], 1.0)
    h = jnp.concatenate([h2_ref[0], h2_ref[1]], axis=1)
    agg = jnp.concatenate([agg2_ref[0], agg2_ref[1]], axis=1)
    hn = jnp.maximum(
        jnp.dot(h + agg / denom, w_ref[...],
                preferred_element_type=jnp.float32) + b_ref[...], 0.0)

    @pl.when(i == 0)
    def _():
        cs_ref[...] = jnp.zeros_like(cs_ref)

    cs_ref[...] += jnp.sum(hn, axis=0, keepdims=True)

    @pl.when(i == N // _BN - 1)
    def _():
        ge = cs_ref[...] * (1.0 / N)
        hidden = jnp.maximum(
            jnp.dot(ge, w1_ref[...], preferred_element_type=jnp.float32)
            + b1_ref[...], 0.0)
        o_ref[...] = jnp.dot(hidden, w2_ref[...],
                             preferred_element_type=jnp.float32) + b2_ref[...]


def _update_head(h2, agg2, deg, w, b, w1, b1, w2, b2):
    return pl.pallas_call(
        _update_head_body,
        grid=(N // _BN,),
        in_specs=[
            pl.BlockSpec((2, _BN, HH), lambda i: (0, i, 0)),
            pl.BlockSpec((2, _BN, HH), lambda i: (0, i, 0)),
            pl.BlockSpec((_BN, DW), lambda i: (i, 0)),
            pl.BlockSpec((H, H), lambda i: (0, 0)),
            pl.BlockSpec((1, H), lambda i: (0, 0)),
            pl.BlockSpec((H, H), lambda i: (0, 0)),
            pl.BlockSpec((1, H), lambda i: (0, 0)),
            pl.BlockSpec((H, 1), lambda i: (0, 0)),
            pl.BlockSpec((1, 1), lambda i: (0, 0)),
        ],
        out_specs=[
            pl.BlockSpec((1, 1), lambda i: (0, 0)),
            pl.BlockSpec((1, H), lambda i: (0, 0)),
        ],
        out_shape=[
            jax.ShapeDtypeStruct((1, 1), jnp.float32),
            jax.ShapeDtypeStruct((1, H), jnp.float32),
        ],
    )(h2, agg2, deg, w, b, w1, b1, w2, b2)


def kernel(x, edge_index, W_emb, b_emb, W_l0, b_l0, W_l1, b_l1,
           W_m1, b_m1, W_m2, b_m2):
    src = edge_index[0]
    dst = edge_index[1]
    zeros = jnp.zeros((RPA, HH), jnp.float32)
    zeros16 = jnp.zeros((RPS, DW), jnp.float32)
    ones = jnp.ones((C, DW), jnp.float32)

    (deg,) = _deg(dst, zeros16, ones)
    h2 = _embed(x, W_emb, b_emb.reshape(1, H))
    (agg0,) = _agg(src, dst, h2.reshape(NC * N, HH), zeros)
    h2, _ = _update(h2, agg0.reshape(NC, N, HH), deg, W_l0, b_l0.reshape(1, H))
    (agg1,) = _agg(src, dst, h2.reshape(NC * N, HH), zeros)
    out, _ = _update_head(h2, agg1.reshape(NC, N, HH), deg, W_l1,
                          b_l1.reshape(1, H), W_m1, b_m1.reshape(1, H),
                          W_m2, b_m2.reshape(1, 1))
    return out.reshape(1)


# speedup vs baseline: 1.0254x; 1.0254x over previous
import jax
import jax.numpy as jnp
from jax import lax
from jax.experimental import pallas as pl
from jax.experimental.pallas import tpu as pltpu
from jax.experimental.pallas import tpu_sc as plsc

N = 50000
E = 800000
F = 111
H = 64
HH = H // 2

NC = 2
NS = 16

RPA = 3128
ACCA = NS * RPA
LASTA = N - 15 * RPA

HALF = N // NC
RPS = 1568
ACC = NS * RPS
LAST = HALF - 15 * RPS
DW = 8

C = 128
K = 3
GE = K * C
EPS = E // NS
NG = EPS // GE
TAIL = EPS - NG * GE

_MESH = plsc.VectorSubcoreMesh(core_axis_name="c", subcore_axis_name="s")
_SC_PARAMS = pltpu.CompilerParams(use_tc_tiling_on_sc=False)


def _agg_body(edge_hbm, hlo_hbm, hhi_hbm, zeros_hbm, alo_hbm, ahi_hbm,
              acc, srcw, dstw, rows, isem0, isem1, *sems):
    gsem, ssem = sems[0:K], sems[K:2 * K]
    c = lax.axis_index("c")
    s = lax.axis_index("s")
    junk = N + s
    ebase = s * EPS

    pltpu.sync_copy(zeros_hbm.at[pl.ds(0, RPA)], acc.at[pl.ds(s * RPA, RPA)])
    plsc.subcore_barrier()


    def load_idx(g, p, sem):
        gbase = ebase + g * GE
        pltpu.async_copy(edge_hbm.at[0, pl.ds(gbase, GE)], srcw.at[p], sem)
        for j in range(K):
            pltpu.async_copy(edge_hbm.at[1, pl.ds(gbase + j * C, C)],
                             dstw.at[lax.rem(g, 3), j], sem)

    def wait_idx(g, p, sem):
        gbase = ebase + g * GE
        pltpu.make_async_copy(edge_hbm.at[0, pl.ds(gbase, GE)], srcw.at[p],
                              sem).wait()
        for j in range(K):
            pltpu.make_async_copy(edge_hbm.at[1, pl.ds(gbase + j * C, C)],
                                  dstw.at[lax.rem(g, 3), j], sem).wait()

    def fire_gathers(p):
        @pl.when(c == 0)
        def _():
            for j in range(K):
                pltpu.async_copy(hlo_hbm.at[srcw.at[p, pl.ds(j * C, C)]],
                                 rows.at[p, j], gsem[j])

        @pl.when(c == 1)
        def _():
            for j in range(K):
                pltpu.async_copy(hhi_hbm.at[srcw.at[p, pl.ds(j * C, C)]],
                                 rows.at[p, j], gsem[j])

    load_idx(0, 0, isem0)
    wait_idx(0, 0, isem0)
    fire_gathers(0)

    def rnd(g, carry):
        p = lax.rem(g, 2)
        pn = lax.rem(g + 1, 2)
        w3 = lax.rem(g, 3)
        w3p = lax.rem(g + 2, 3)

        @pl.when(g >= 1)
        def _():
            for j in range(K):
                pltpu.make_async_copy(rows.at[pn, j], acc.at[dstw.at[w3p, j]],
                                      ssem[j]).wait()

        @pl.when(jnp.logical_and(g + 1 < NG, pn == 0))
        def _():
            load_idx(g + 1, 0, isem0)

        @pl.when(jnp.logical_and(g + 1 < NG, pn == 1))
        def _():
            load_idx(g + 1, 1, isem1)

        for j in range(K):
            pltpu.make_async_copy(hlo_hbm.at[srcw.at[p, pl.ds(j * C, C)]],
                                  rows.at[p, j], gsem[j]).wait()
            pltpu.async_copy(rows.at[p, j], acc.at[dstw.at[w3, j]], ssem[j],
                             add=True)

        @pl.when(jnp.logical_and(g + 1 < NG, pn == 0))
        def _():
            wait_idx(g + 1, 0, isem0)

        @pl.when(jnp.logical_and(g + 1 < NG, pn == 1))
        def _():
            wait_idx(g + 1, 1, isem1)

        @pl.when(g + 1 < NG)
        def _():
            fire_gathers(pn)

        return carry

    lax.fori_loop(0, NG, rnd, 0)

    for j in range(K):
        pltpu.make_async_copy(rows.at[(NG - 1) % 2, j],
                              acc.at[dstw.at[(NG - 1) % 3, j]],
                              ssem[j]).wait()

    toff = ebase + NG * GE
    pltpu.sync_copy(edge_hbm.at[0, pl.ds(toff, TAIL)],
                    srcw.at[0, pl.ds(0, TAIL)])
    pltpu.sync_copy(edge_hbm.at[1, pl.ds(toff, TAIL)],
                    dstw.at[0, 0, pl.ds(0, TAIL)])
    zero16 = jnp.zeros((16,), jnp.int32)
    for q in range(TAIL // 16, C // 16):
        srcw[0, pl.ds(q * 16, 16)] = zero16
        dstw[0, 0, pl.ds(q * 16, 16)] = zero16 + junk

    @pl.when(c == 0)
    def _():
        pltpu.async_copy(hlo_hbm.at[srcw.at[0, pl.ds(0, C)]], rows.at[0, 0],
                         gsem[0]).wait()

    @pl.when(c == 1)
    def _():
        pltpu.async_copy(hhi_hbm.at[srcw.at[0, pl.ds(0, C)]], rows.at[0, 0],
                         gsem[0]).wait()

    pltpu.sync_copy(rows.at[0, 0], acc.at[dstw.at[0, 0]], add=True)

    plsc.subcore_barrier()

    @pl.when(jnp.logical_and(c == 0, s < NS - 1))
    def _():
        pltpu.sync_copy(acc.at[pl.ds(s * RPA, RPA)],
                        alo_hbm.at[pl.ds(s * RPA, RPA)])

    @pl.when(jnp.logical_and(c == 0, s == NS - 1))
    def _():
        pltpu.sync_copy(acc.at[pl.ds((NS - 1) * RPA, LASTA)],
                        alo_hbm.at[pl.ds((NS - 1) * RPA, LASTA)])

    @pl.when(jnp.logical_and(c == 1, s < NS - 1))
    def _():
        pltpu.sync_copy(acc.at[pl.ds(s * RPA, RPA)],
                        ahi_hbm.at[pl.ds(s * RPA, RPA)])

    @pl.when(jnp.logical_and(c == 1, s == NS - 1))
    def _():
        pltpu.sync_copy(acc.at[pl.ds((NS - 1) * RPA, LASTA)],
                        ahi_hbm.at[pl.ds((NS - 1) * RPA, LASTA)])


_agg = pl.kernel(
    _agg_body,
    out_type=(
        jax.ShapeDtypeStruct((N, HH), jnp.float32),
        jax.ShapeDtypeStruct((N, HH), jnp.float32),
    ),
    mesh=_MESH,
    scratch_types=(
        pltpu.VMEM_SHARED((ACCA, HH), jnp.float32),
        pltpu.VMEM((2, GE), jnp.int32),
        pltpu.VMEM((3, K, C), jnp.int32),
        pltpu.VMEM((2, K, C, HH), jnp.float32),
    ) + (pltpu.SemaphoreType.DMA,) * (2 + 2 * K),
    compiler_params=_SC_PARAMS,
)


def _deg_body(edge_hbm, zeros16_hbm, ones_hbm, deg_hbm,
              dacc, dstw, dloc, onesv, isem0, isem1, *ssem):
    c = lax.axis_index("c")
    s = lax.axis_index("s")
    base_node = c * HALF
    junk = HALF + s
    ebase = s * EPS

    pltpu.sync_copy(zeros16_hbm, dacc.at[pl.ds(s * RPS, RPS)])
    pltpu.sync_copy(ones_hbm, onesv)
    plsc.subcore_barrier()

    pltpu.async_copy(edge_hbm.at[1, pl.ds(ebase, GE)], dstw.at[0], isem0)

    def compute_dloc(p, j, nvec):
        for q in range(nvec):
            d = dstw[p, pl.ds(j * C + q * 16, 16)]
            loc = d - base_node
            ok = (loc >= 0) & (loc < HALF)
            dloc[j, pl.ds(q * 16, 16)] = jnp.where(ok, loc, junk)

    def rnd(g, carry):
        gbase = ebase + g * GE
        p = lax.rem(g, 2)

        @pl.when(jnp.logical_and(g + 1 < NG, p == 0))
        def _():
            pltpu.async_copy(edge_hbm.at[1, pl.ds(gbase + GE, GE)],
                             dstw.at[1], isem1)

        @pl.when(jnp.logical_and(g + 1 < NG, p == 1))
        def _():
            pltpu.async_copy(edge_hbm.at[1, pl.ds(gbase + GE, GE)],
                             dstw.at[0], isem0)

        @pl.when(g >= 1)
        def _():
            for j in range(K):
                pltpu.make_async_copy(onesv, dacc.at[dloc.at[j]],
                                      ssem[j]).wait()

        @pl.when(p == 0)
        def _():
            pltpu.make_async_copy(edge_hbm.at[1, pl.ds(gbase, GE)],
                                  dstw.at[0], isem0).wait()

        @pl.when(p == 1)
        def _():
            pltpu.make_async_copy(edge_hbm.at[1, pl.ds(gbase, GE)],
                                  dstw.at[1], isem1).wait()

        for j in range(K):
            compute_dloc(p, j, C // 16)
        for j in range(K):
            pltpu.async_copy(onesv, dacc.at[dloc.at[j]], ssem[j], add=True)
        return carry

    lax.fori_loop(0, NG, rnd, 0)
    for j in range(K):
        pltpu.make_async_copy(onesv, dacc.at[dloc.at[j]], ssem[j]).wait()

    toff = ebase + NG * GE
    pltpu.sync_copy(edge_hbm.at[1, pl.ds(toff, TAIL)],
                    dstw.at[0, pl.ds(0, TAIL)])
    compute_dloc(0, 0, TAIL // 16)
    zero16 = jnp.zeros((16,), jnp.int32)
    for q in range(TAIL // 16, C // 16):
        dloc[0, pl.ds(q * 16, 16)] = zero16 + junk
    pltpu.sync_copy(onesv, dacc.at[dloc.at[0]], add=True)

    plsc.subcore_barrier()

    @pl.when(s < NS - 1)
    def _():
        pltpu.sync_copy(dacc.at[pl.ds(s * RPS, RPS)],
                        deg_hbm.at[pl.ds(c * HALF + s * RPS, RPS)])

    @pl.when(s == NS - 1)
    def _():
        pltpu.sync_copy(dacc.at[pl.ds((NS - 1) * RPS, LAST)],
                        deg_hbm.at[pl.ds(c * HALF + (NS - 1) * RPS, LAST)])


_deg = pl.kernel(
    _deg_body,
    out_type=(jax.ShapeDtypeStruct((N, DW), jnp.float32),),
    mesh=_MESH,
    scratch_types=(
        pltpu.VMEM_SHARED((ACC, DW), jnp.float32),
        pltpu.VMEM((2, GE), jnp.int32),
        pltpu.VMEM((K, C), jnp.int32),
        pltpu.VMEM((C, DW), jnp.float32),
    ) + (pltpu.SemaphoreType.DMA,) * (2 + K),
    compiler_params=_SC_PARAMS,
)

_BN = 2000


def _embed_body(x_ref, w_ref, b_ref, olo_ref, ohi_ref):
    hn = jnp.dot(x_ref[...], w_ref[...],
                 preferred_element_type=jnp.float32) + b_ref[...]
    olo_ref[...] = hn[:, :HH]
    ohi_ref[...] = hn[:, HH:]


def _embed(x, w, b):
    return pl.pallas_call(
        _embed_body,
        grid=(N // _BN,),
        in_specs=[
            pl.BlockSpec((_BN, F), lambda i: (i, 0)),
            pl.BlockSpec((F, H), lambda i: (0, 0)),
            pl.BlockSpec((1, H), lambda i: (0, 0)),
        ],
        out_specs=[
            pl.BlockSpec((_BN, HH), lambda i: (i, 0)),
            pl.BlockSpec((_BN, HH), lambda i: (i, 0)),
        ],
        out_shape=[
            jax.ShapeDtypeStruct((N, HH), jnp.float32),
            jax.ShapeDtypeStruct((N, HH), jnp.float32),
        ],
    )(x, w, b)


def _update_body(hlo_ref, hhi_ref, alo_ref, ahi_ref, deg_ref, w_ref, b_ref,
                 olo_ref, ohi_ref, cs_ref):
    i = pl.program_id(0)
    denom = jnp.maximum(deg_ref[:, 0:1], 1.0)
    h = jnp.concatenate([hlo_ref[...], hhi_ref[...]], axis=1)
    agg = jnp.concatenate([alo_ref[...], ahi_ref[...]], axis=1)
    hn = jnp.maximum(
        jnp.dot(h + agg / denom, w_ref[...],
                preferred_element_type=jnp.float32) + b_ref[...], 0.0)
    olo_ref[...] = hn[:, :HH]
    ohi_ref[...] = hn[:, HH:]

    @pl.when(i == 0)
    def _():
        cs_ref[...] = jnp.zeros_like(cs_ref)

    cs_ref[...] += jnp.sum(hn, axis=0, keepdims=True)


def _update(hlo, hhi, alo, ahi, deg, w, b):
    return pl.pallas_call(
        _update_body,
        grid=(N // _BN,),
        in_specs=[
            pl.BlockSpec((_BN, HH), lambda i: (i, 0)),
            pl.BlockSpec((_BN, HH), lambda i: (i, 0)),
            pl.BlockSpec((_BN, HH), lambda i: (i, 0)),
            pl.BlockSpec((_BN, HH), lambda i: (i, 0)),
            pl.BlockSpec((_BN, DW), lambda i: (i, 0)),
            pl.BlockSpec((H, H), lambda i: (0, 0)),
            pl.BlockSpec((1, H), lambda i: (0, 0)),
        ],
        out_specs=[
            pl.BlockSpec((_BN, HH), lambda i: (i, 0)),
            pl.BlockSpec((_BN, HH), lambda i: (i, 0)),
            pl.BlockSpec((1, H), lambda i: (0, 0)),
        ],
        out_shape=[
            jax.ShapeDtypeStruct((N, HH), jnp.float32),
            jax.ShapeDtypeStruct((N, HH), jnp.float32),
            jax.ShapeDtypeStruct((1, H), jnp.float32),
        ],
    )(hlo, hhi, alo, ahi, deg, w, b)


def _update_head_body(hlo_ref, hhi_ref, alo_ref, ahi_ref, deg_ref, w_ref,
                      b_ref, w1_ref, b1_ref, w2_ref, b2_ref, o_ref, cs_ref):
    i = pl.program_id(0)
    denom = jnp.maximum(deg_ref[:, 0:1], 1.0)
    h = jnp.concatenate([hlo_ref[...], hhi_ref[...]], axis=1)
    agg = jnp.concatenate([alo_ref[...], ahi_ref[...]], axis=1)
    hn = jnp.maximum(
        jnp.dot(h + agg / denom, w_ref[...],
                preferred_element_type=jnp.float32) + b_ref[...], 0.0)

    @pl.when(i == 0)
    def _():
        cs_ref[...] = jnp.zeros_like(cs_ref)

    cs_ref[...] += jnp.sum(hn, axis=0, keepdims=True)

    @pl.when(i == N // _BN - 1)
    def _():
        ge = cs_ref[...] * (1.0 / N)
        hidden = jnp.maximum(
            jnp.dot(ge, w1_ref[...], preferred_element_type=jnp.float32)
            + b1_ref[...], 0.0)
        o_ref[...] = jnp.dot(hidden, w2_ref[...],
                             preferred_element_type=jnp.float32) + b2_ref[...]


def _update_head(hlo, hhi, alo, ahi, deg, w, b, w1, b1, w2, b2):
    return pl.pallas_call(
        _update_head_body,
        grid=(N // _BN,),
        in_specs=[
            pl.BlockSpec((_BN, HH), lambda i: (i, 0)),
            pl.BlockSpec((_BN, HH), lambda i: (i, 0)),
            pl.BlockSpec((_BN, HH), lambda i: (i, 0)),
            pl.BlockSpec((_BN, HH), lambda i: (i, 0)),
            pl.BlockSpec((_BN, DW), lambda i: (i, 0)),
            pl.BlockSpec((H, H), lambda i: (0, 0)),
            pl.BlockSpec((1, H), lambda i: (0, 0)),
            pl.BlockSpec((H, H), lambda i: (0, 0)),
            pl.BlockSpec((1, H), lambda i: (0, 0)),
            pl.BlockSpec((H, 1), lambda i: (0, 0)),
            pl.BlockSpec((1, 1), lambda i: (0, 0)),
        ],
        out_specs=[
            pl.BlockSpec((1, 1), lambda i: (0, 0)),
            pl.BlockSpec((1, H), lambda i: (0, 0)),
        ],
        out_shape=[
            jax.ShapeDtypeStruct((1, 1), jnp.float32),
            jax.ShapeDtypeStruct((1, H), jnp.float32),
        ],
    )(hlo, hhi, alo, ahi, deg, w, b, w1, b1, w2, b2)


def kernel(x, edge_index, W_emb, b_emb, W_l0, b_l0, W_l1, b_l1,
           W_m1, b_m1, W_m2, b_m2):
    zeros = jnp.zeros((RPA, HH), jnp.float32)
    zeros16 = jnp.zeros((RPS, DW), jnp.float32)
    ones = jnp.ones((C, DW), jnp.float32)

    (deg,) = _deg(edge_index, zeros16, ones)
    hlo, hhi = _embed(x, W_emb, b_emb.reshape(1, H))
    alo, ahi = _agg(edge_index, hlo, hhi, zeros)
    hlo, hhi, _ = _update(hlo, hhi, alo, ahi, deg, W_l0, b_l0.reshape(1, H))
    alo, ahi = _agg(edge_index, hlo, hhi, zeros)
    out, _ = _update_head(hlo, hhi, alo, ahi, deg, W_l1, b_l1.reshape(1, H),
                          W_m1, b_m1.reshape(1, H), W_m2, b_m2.reshape(1, 1))
    return out.reshape(1)
